# baseline (speedup 1.0000x reference)
import jax
import jax.numpy as jnp
from jax import lax
from jax.experimental import pallas as pl
from jax.experimental.pallas import tpu as pltpu
from jax.experimental.pallas import tpu_sc as plsc

NUM_CLASSES_ = 100000
D_ = 128
B_ = 16384
NW_ = 32
ROWS_PER_W = B_ // NW_
CHUNK = 64
NCHUNK = ROWS_PER_W // CHUNK
NBUF = 3
L_ = 16
NJ = D_ // L_


def _rsqrt16(x):
    i = plsc.bitcast(x, jnp.int32)
    i = 0x5F3759DF - (i >> 1)
    y = plsc.bitcast(i, jnp.float32)
    for _ in range(3):
        y = y * (1.5 - 0.5 * x * y * y)
    return y


def _compute_chunk(bl, br, ob, posv):
    @plsc.parallel_loop(0, CHUNK, unroll=4)
    def row(r):
        e = [bl[r, pl.ds(j * L_, L_)] + br[r, pl.ds(j * L_, L_)] + posv[j]
             for j in range(NJ)]
        ss = e[0] * e[0]
        for j in range(1, NJ):
            ss = ss + e[j] * e[j]
        tot = jnp.sum(ss)
        tv = jnp.broadcast_to(tot, (L_,))
        rinv = jnp.minimum(_rsqrt16(tv), 1e12)
        for j in range(NJ):
            ob[r, pl.ds(j * L_, L_)] = e[j] * rinv


def _body(left2d, right2d, table, pos_l, pos_r, out, idxl_v, idxr_v,
          bufl0, bufl1, bufl2, bufr0, bufr1, bufr2, obuf0, obuf1, obuf2,
          poslv, posrv, gl0, gl1, gl2, gr0, gr1, gr2, so0, so1, so2):
    bufl = (bufl0, bufl1, bufl2)
    bufr = (bufr0, bufr1, bufr2)
    obuf = (obuf0, obuf1, obuf2)
    gl = (gl0, gl1, gl2)
    gr = (gr0, gr1, gr2)
    so = (so0, so1, so2)

    wid = lax.axis_index("s") * 2 + lax.axis_index("c")
    base = wid * ROWS_PER_W

    pltpu.sync_copy(left2d.at[pl.ds(wid * NCHUNK, NCHUNK)], idxl_v)
    pltpu.sync_copy(right2d.at[pl.ds(wid * NCHUNK, NCHUNK)], idxr_v)
    pltpu.sync_copy(pos_l, poslv)
    pltpu.sync_copy(pos_r, posrv)

    posv = [poslv[pl.ds(j * L_, L_)] + posrv[pl.ds(j * L_, L_)]
            for j in range(NJ)]

    def issue_gather(c):
        b = c % NBUF
        cl = pltpu.async_copy(table.at[idxl_v.at[c]], bufl[b], gl[b])
        cr = pltpu.async_copy(table.at[idxr_v.at[c]], bufr[b], gr[b])
        return cl, cr

    descs = [None] * NCHUNK
    odesc = [None] * NCHUNK
    descs[0] = issue_gather(0)
    descs[1] = issue_gather(1)
    for c in range(NCHUNK):
        b = c % NBUF
        if c + 2 < NCHUNK:
            descs[c + 2] = issue_gather(c + 2)
        descs[c][0].wait()
        descs[c][1].wait()
        if c >= NBUF:
            odesc[c - NBUF].wait()
        _compute_chunk(bufl[b], bufr[b], obuf[b], posv)
        odesc[c] = pltpu.async_copy(
            obuf[b], out.at[pl.ds(base + c * CHUNK, CHUNK)], so[b])
    for c in range(NCHUNK - NBUF, NCHUNK):
        odesc[c].wait()


def kernel(left_idx, right_idx, class_emb, pos_left, pos_right):
    left2d = left_idx.reshape(B_ // CHUNK, CHUNK).astype(jnp.int32)
    right2d = right_idx.reshape(B_ // CHUNK, CHUNK).astype(jnp.int32)
    k = pl.kernel(
        _body,
        out_type=jax.ShapeDtypeStruct((B_, D_), jnp.float32),
        mesh=plsc.VectorSubcoreMesh(core_axis_name="c", subcore_axis_name="s"),
        compiler_params=pltpu.CompilerParams(needs_layout_passes=False),
        scratch_types=(
            [pltpu.VMEM((NCHUNK, CHUNK), jnp.int32)] * 2
            + [pltpu.VMEM((CHUNK, D_), jnp.float32)] * (3 * NBUF)
            + [pltpu.VMEM((D_,), jnp.float32)] * 2
            + [pltpu.SemaphoreType.DMA] * (3 * NBUF)
        ),
    )
    return k(left2d, right2d, class_emb, pos_left, pos_right)


# speedup vs baseline: 1.0282x; 1.0282x over previous
import jax
import jax.numpy as jnp
from jax import lax
from jax.experimental import pallas as pl
from jax.experimental.pallas import tpu as pltpu
from jax.experimental.pallas import tpu_sc as plsc

NUM_CLASSES_ = 100000
D_ = 128
B_ = 16384
NW_ = 32
ROWS_PER_W = B_ // NW_
CHUNK = 128
NCHUNK = ROWS_PER_W // CHUNK
L_ = 16
NJ = D_ // L_


def _rsqrt16(x):
    i = plsc.bitcast(x, jnp.int32)
    i = 0x5F3759DF - (i >> 1)
    y = plsc.bitcast(i, jnp.float32)
    hx = 0.5 * x
    for _ in range(2):
        y = y * (1.5 - hx * y * y)
    return y


def _compute_chunk(bl, br, ob, posv):
    @plsc.parallel_loop(0, CHUNK, unroll=8)
    def row(r):
        e = [bl[r, pl.ds(j * L_, L_)] + br[r, pl.ds(j * L_, L_)] + posv[j]
             for j in range(NJ)]
        sq = [e[j] * e[j] for j in range(NJ)]
        s01 = sq[0] + sq[1]
        s23 = sq[2] + sq[3]
        s45 = sq[4] + sq[5]
        s67 = sq[6] + sq[7]
        ss = (s01 + s23) + (s45 + s67)
        tot = jnp.sum(ss)
        tv = jnp.broadcast_to(tot, (L_,))
        rinv = jnp.minimum(_rsqrt16(tv), 1e12)
        for j in range(NJ):
            ob[r, pl.ds(j * L_, L_)] = e[j] * rinv


def _body(left2d, right2d, table, pos_l, pos_r, out,
          idxl_v, idxr_v, bufl0, bufl1, bufr0, bufr1, obuf0, obuf1,
          poslv, posrv, gl0, gl1, gr0, gr1, so0, so1):
    bufl = (bufl0, bufl1)
    bufr = (bufr0, bufr1)
    obuf = (obuf0, obuf1)
    gl = (gl0, gl1)
    gr = (gr0, gr1)
    so = (so0, so1)

    wid = lax.axis_index("s") * 2 + lax.axis_index("c")
    base = wid * ROWS_PER_W

    pltpu.sync_copy(left2d.at[pl.ds(wid * NCHUNK, NCHUNK)], idxl_v)
    pltpu.sync_copy(right2d.at[pl.ds(wid * NCHUNK, NCHUNK)], idxr_v)
    pltpu.sync_copy(pos_l, poslv)
    pltpu.sync_copy(pos_r, posrv)

    posv = [poslv[pl.ds(j * L_, L_)] + posrv[pl.ds(j * L_, L_)]
            for j in range(NJ)]

    def issue_gather(c):
        b = c & 1
        cl = pltpu.async_copy(table.at[idxl_v.at[c]], bufl[b], gl[b])
        cr = pltpu.async_copy(table.at[idxr_v.at[c]], bufr[b], gr[b])
        return cl, cr

    descs = [None] * NCHUNK
    odesc = [None] * NCHUNK
    descs[0] = issue_gather(0)
    for c in range(NCHUNK):
        b = c & 1
        if c + 1 < NCHUNK:
            descs[c + 1] = issue_gather(c + 1)
        descs[c][0].wait()
        descs[c][1].wait()
        if c >= 2:
            odesc[c - 2].wait()
        _compute_chunk(bufl[b], bufr[b], obuf[b], posv)
        odesc[c] = pltpu.async_copy(
            obuf[b], out.at[pl.ds(base + c * CHUNK, CHUNK)], so[b])
    odesc[NCHUNK - 2].wait()
    odesc[NCHUNK - 1].wait()


def kernel(left_idx, right_idx, class_emb, pos_left, pos_right):
    left2d = left_idx.reshape(B_ // CHUNK, CHUNK).astype(jnp.int32)
    right2d = right_idx.reshape(B_ // CHUNK, CHUNK).astype(jnp.int32)
    k = pl.kernel(
        _body,
        out_type=jax.ShapeDtypeStruct((B_, D_), jnp.float32),
        mesh=plsc.VectorSubcoreMesh(core_axis_name="c", subcore_axis_name="s"),
        compiler_params=pltpu.CompilerParams(needs_layout_passes=False),
        scratch_types=(
            [pltpu.VMEM((NCHUNK, CHUNK), jnp.int32)] * 2
            + [pltpu.VMEM((CHUNK, D_), jnp.float32)] * 6
            + [pltpu.VMEM((D_,), jnp.float32)] * 2
            + [pltpu.SemaphoreType.DMA] * 6
        ),
    )
    return k(left2d, right2d, class_emb, pos_left, pos_right)


# speedup vs baseline: 1.0965x; 1.0663x over previous
import jax
import jax.numpy as jnp
from jax import lax
from jax.experimental import pallas as pl
from jax.experimental.pallas import tpu as pltpu
from jax.experimental.pallas import tpu_sc as plsc

NUM_CLASSES_ = 100000
D_ = 128
B_ = 16384
NW_ = 32
ROWS_PER_W = B_ // NW_
CHUNK = 128
NCHUNK = ROWS_PER_W // CHUNK
NBUF = 3
L_ = 16
NJ = D_ // L_


def _rsqrt16(x):
    i = plsc.bitcast(x, jnp.int32)
    i = 0x5F3759DF - (i >> 1)
    y = plsc.bitcast(i, jnp.float32)
    hx = 0.5 * x
    for _ in range(2):
        y = y * (1.5 - hx * y * y)
    return y


def _compute_chunk(bg, ob, posv):
    @plsc.parallel_loop(0, CHUNK, unroll=4)
    def row(r):
        e = [bg[r, pl.ds(j * L_, L_)] + posv[j] for j in range(NJ)]
        sq = [e[j] * e[j] for j in range(NJ)]
        s01 = sq[0] + sq[1]
        s23 = sq[2] + sq[3]
        s45 = sq[4] + sq[5]
        s67 = sq[6] + sq[7]
        ss = (s01 + s23) + (s45 + s67)
        tot = jnp.sum(ss)
        tv = jnp.broadcast_to(tot, (L_,))
        rinv = jnp.minimum(_rsqrt16(tv), 1e12)
        for j in range(NJ):
            ob[r, pl.ds(j * L_, L_)] = e[j] * rinv


def _body(left2d, right2d, table, pos_l, pos_r, out,
          idxl_v, idxr_v, bufg0, bufg1, bufg2, obuf0, obuf1,
          poslv, posrv, g0, g1, g2, so0, so1):
    bufg = (bufg0, bufg1, bufg2)
    obuf = (obuf0, obuf1)
    gsem = (g0, g1, g2)
    so = (so0, so1)

    wid = lax.axis_index("s") * 2 + lax.axis_index("c")
    base = wid * ROWS_PER_W

    pltpu.sync_copy(left2d.at[pl.ds(wid * NCHUNK, NCHUNK)], idxl_v)
    pltpu.sync_copy(right2d.at[pl.ds(wid * NCHUNK, NCHUNK)], idxr_v)
    pltpu.sync_copy(pos_l, poslv)
    pltpu.sync_copy(pos_r, posrv)

    posv = [poslv[pl.ds(j * L_, L_)] + posrv[pl.ds(j * L_, L_)]
            for j in range(NJ)]

    def issue_left(c):
        b = c % NBUF
        return pltpu.async_copy(table.at[idxl_v.at[c]], bufg[b], gsem[b])

    def issue_right_add(c):
        b = c % NBUF
        return pltpu.async_copy(table.at[idxr_v.at[c]], bufg[b], gsem[b],
                                add=True)

    ldesc = [None] * NCHUNK
    adesc = [None] * NCHUNK
    odesc = [None] * NCHUNK
    ldesc[0] = issue_left(0)
    ldesc[1] = issue_left(1)
    ldesc[0].wait()
    adesc[0] = issue_right_add(0)
    for c in range(NCHUNK):
        b = c % NBUF
        if c + 2 < NCHUNK:
            ldesc[c + 2] = issue_left(c + 2)
        adesc[c].wait()
        if c + 1 < NCHUNK:
            ldesc[c + 1].wait()
            adesc[c + 1] = issue_right_add(c + 1)
        if c >= 2:
            odesc[c - 2].wait()
        _compute_chunk(bufg[b], obuf[c & 1], posv)
        odesc[c] = pltpu.async_copy(
            obuf[c & 1], out.at[pl.ds(base + c * CHUNK, CHUNK)], so[c & 1])
    odesc[NCHUNK - 2].wait()
    odesc[NCHUNK - 1].wait()


def kernel(left_idx, right_idx, class_emb, pos_left, pos_right):
    left2d = left_idx.reshape(B_ // CHUNK, CHUNK).astype(jnp.int32)
    right2d = right_idx.reshape(B_ // CHUNK, CHUNK).astype(jnp.int32)
    k = pl.kernel(
        _body,
        out_type=jax.ShapeDtypeStruct((B_, D_), jnp.float32),
        mesh=plsc.VectorSubcoreMesh(core_axis_name="c", subcore_axis_name="s"),
        compiler_params=pltpu.CompilerParams(needs_layout_passes=False),
        scratch_types=(
            [pltpu.VMEM((NCHUNK, CHUNK), jnp.int32)] * 2
            + [pltpu.VMEM((CHUNK, D_), jnp.float32)] * (NBUF + 2)
            + [pltpu.VMEM((D_,), jnp.float32)] * 2
            + [pltpu.SemaphoreType.DMA] * (NBUF + 2)
        ),
    )
    return k(left2d, right2d, class_emb, pos_left, pos_right)


# speedup vs baseline: 1.1457x; 1.0449x over previous
import jax
import jax.numpy as jnp
from jax import lax
from jax.experimental import pallas as pl
from jax.experimental.pallas import tpu as pltpu
from jax.experimental.pallas import tpu_sc as plsc

NUM_CLASSES_ = 100000
D_ = 128
B_ = 16384
NW_ = 32
ROWS_PER_W = B_ // NW_
CHUNK = 128
NCHUNK = ROWS_PER_W // CHUNK
NBUF = 3
L_ = 16
NJ = D_ // L_


def _rsqrt16(x):
    i = plsc.bitcast(x, jnp.int32)
    i = 0x5F3759DF - (i >> 1)
    y = plsc.bitcast(i, jnp.float32)
    hx = 0.5 * x
    for _ in range(2):
        y = y * (1.5 - hx * y * y)
    return y


def _compute_chunk(bg, ob, posv):
    @plsc.parallel_loop(0, CHUNK, unroll=4)
    def row(r):
        e = [bg[r, pl.ds(j * L_, L_)] + posv[j] for j in range(NJ)]
        sq = [e[j] * e[j] for j in range(NJ)]
        s01 = sq[0] + sq[1]
        s23 = sq[2] + sq[3]
        s45 = sq[4] + sq[5]
        s67 = sq[6] + sq[7]
        ss = (s01 + s23) + (s45 + s67)
        tot = jnp.sum(ss)
        tv = jnp.broadcast_to(tot, (L_,))
        rinv = jnp.minimum(_rsqrt16(tv), 1e12)
        for j in range(NJ):
            ob[r, pl.ds(j * L_, L_)] = e[j] * rinv


def _body(left2d, right2d, table, pos_l, pos_r, out,
          idxl_v, idxr_v, bufg0, bufg1, bufg2, obuf0, obuf1,
          poslv, posrv, g0, g1, g2, so0, so1, stg):
    bufg = (bufg0, bufg1, bufg2)
    obuf = (obuf0, obuf1)
    gsem = (g0, g1, g2)
    so = (so0, so1)

    wid = lax.axis_index("s") * 2 + lax.axis_index("c")
    base = wid * ROWS_PER_W

    s1 = pltpu.async_copy(left2d.at[pl.ds(wid * NCHUNK, NCHUNK)], idxl_v, stg)
    s2 = pltpu.async_copy(right2d.at[pl.ds(wid * NCHUNK, NCHUNK)], idxr_v, stg)
    s3 = pltpu.async_copy(pos_l, poslv, stg)
    s4 = pltpu.async_copy(pos_r, posrv, stg)
    def issue_left(c):
        b = c % NBUF
        return pltpu.async_copy(table.at[idxl_v.at[c]], bufg[b], gsem[b])

    def issue_right_add(c):
        b = c % NBUF
        return pltpu.async_copy(table.at[idxr_v.at[c]], bufg[b], gsem[b],
                                add=True)

    ldesc = [None] * NCHUNK
    adesc = [None] * NCHUNK
    odesc = [None] * NCHUNK
    s1.wait()
    ldesc[0] = issue_left(0)
    ldesc[1] = issue_left(1)
    s2.wait()
    ldesc[0].wait()
    adesc[0] = issue_right_add(0)
    s3.wait()
    s4.wait()
    posv = [poslv[pl.ds(j * L_, L_)] + posrv[pl.ds(j * L_, L_)]
            for j in range(NJ)]
    for c in range(NCHUNK):
        b = c % NBUF
        if c + 2 < NCHUNK:
            ldesc[c + 2] = issue_left(c + 2)
        adesc[c].wait()
        if c + 1 < NCHUNK:
            ldesc[c + 1].wait()
            adesc[c + 1] = issue_right_add(c + 1)
        if c >= 2:
            odesc[c - 2].wait()
        _compute_chunk(bufg[b], obuf[c & 1], posv)
        odesc[c] = pltpu.async_copy(
            obuf[c & 1], out.at[pl.ds(base + c * CHUNK, CHUNK)], so[c & 1])
    odesc[NCHUNK - 2].wait()
    odesc[NCHUNK - 1].wait()


def kernel(left_idx, right_idx, class_emb, pos_left, pos_right):
    left2d = left_idx.reshape(B_ // CHUNK, CHUNK).astype(jnp.int32)
    right2d = right_idx.reshape(B_ // CHUNK, CHUNK).astype(jnp.int32)
    k = pl.kernel(
        _body,
        out_type=jax.ShapeDtypeStruct((B_, D_), jnp.float32),
        mesh=plsc.VectorSubcoreMesh(core_axis_name="c", subcore_axis_name="s"),
        compiler_params=pltpu.CompilerParams(needs_layout_passes=False),
        scratch_types=(
            [pltpu.VMEM((NCHUNK, CHUNK), jnp.int32)] * 2
            + [pltpu.VMEM((CHUNK, D_), jnp.float32)] * (NBUF + 2)
            + [pltpu.VMEM((D_,), jnp.float32)] * 2
            + [pltpu.SemaphoreType.DMA] * (NBUF + 3)
        ),
    )
    return k(left2d, right2d, class_emb, pos_left, pos_right)


# speedup vs baseline: 1.1598x; 1.0124x over previous
import jax
import jax.numpy as jnp
from jax import lax
from jax.experimental import pallas as pl
from jax.experimental.pallas import tpu as pltpu
from jax.experimental.pallas import tpu_sc as plsc

NUM_CLASSES_ = 100000
D_ = 128
B_ = 16384
NW_ = 32
ROWS_PER_W = B_ // NW_
CS = (64, 128, 128, 128, 64)
OFF = (0, 64, 192, 320, 448)
NCHUNK = len(CS)
NBUF = 3
MAXC = 128
L_ = 16
NJ = D_ // L_


def _rsqrt16(x):
    i = plsc.bitcast(x, jnp.int32)
    i = 0x5F3759DF - (i >> 1)
    y = plsc.bitcast(i, jnp.float32)
    hx = 0.5 * x
    for _ in range(2):
        y = y * (1.5 - hx * y * y)
    return y


def _compute_chunk(bg, ob, posv, n):
    @plsc.parallel_loop(0, n, unroll=4)
    def row(r):
        e = [bg[r, pl.ds(j * L_, L_)] + posv[j] for j in range(NJ)]
        sq = [e[j] * e[j] for j in range(NJ)]
        s01 = sq[0] + sq[1]
        s23 = sq[2] + sq[3]
        s45 = sq[4] + sq[5]
        s67 = sq[6] + sq[7]
        ss = (s01 + s23) + (s45 + s67)
        tot = jnp.sum(ss)
        tv = jnp.broadcast_to(tot, (L_,))
        rinv = jnp.minimum(_rsqrt16(tv), 1e12)
        for j in range(NJ):
            ob[r, pl.ds(j * L_, L_)] = e[j] * rinv


def _body(lidx, ridx, table, pos_l, pos_r, out,
          idxl_v, idxr_v, bufg0, bufg1, bufg2, obuf0, obuf1,
          poslv, posrv, g0, g1, g2, so0, so1, stg):
    bufg = (bufg0, bufg1, bufg2)
    obuf = (obuf0, obuf1)
    gsem = (g0, g1, g2)
    so = (so0, so1)

    wid = lax.axis_index("s") * 2 + lax.axis_index("c")
    base = wid * ROWS_PER_W

    s1 = pltpu.async_copy(lidx.at[pl.ds(base, ROWS_PER_W)], idxl_v, stg)
    s2 = pltpu.async_copy(ridx.at[pl.ds(base, ROWS_PER_W)], idxr_v, stg)
    s3 = pltpu.async_copy(pos_l, poslv, stg)
    s4 = pltpu.async_copy(pos_r, posrv, stg)

    def issue_left(c):
        b = c % NBUF
        return pltpu.async_copy(
            table.at[idxl_v.at[pl.ds(OFF[c], CS[c])]],
            bufg[b].at[pl.ds(0, CS[c])], gsem[b])

    def issue_right_add(c):
        b = c % NBUF
        return pltpu.async_copy(
            table.at[idxr_v.at[pl.ds(OFF[c], CS[c])]],
            bufg[b].at[pl.ds(0, CS[c])], gsem[b], add=True)

    ldesc = [None] * NCHUNK
    adesc = [None] * NCHUNK
    odesc = [None] * NCHUNK
    s1.wait()
    ldesc[0] = issue_left(0)
    ldesc[1] = issue_left(1)
    s2.wait()
    ldesc[0].wait()
    adesc[0] = issue_right_add(0)
    s3.wait()
    s4.wait()
    posv = [poslv[pl.ds(j * L_, L_)] + posrv[pl.ds(j * L_, L_)]
            for j in range(NJ)]
    for c in range(NCHUNK):
        b = c % NBUF
        if c + 2 < NCHUNK:
            ldesc[c + 2] = issue_left(c + 2)
        adesc[c].wait()
        if c + 1 < NCHUNK:
            ldesc[c + 1].wait()
            adesc[c + 1] = issue_right_add(c + 1)
        if c >= 2:
            odesc[c - 2].wait()
        _compute_chunk(bufg[b], obuf[c & 1], posv, CS[c])
        odesc[c] = pltpu.async_copy(
            obuf[c & 1].at[pl.ds(0, CS[c])],
            out.at[pl.ds(base + OFF[c], CS[c])], so[c & 1])
    odesc[NCHUNK - 2].wait()
    odesc[NCHUNK - 1].wait()


def kernel(left_idx, right_idx, class_emb, pos_left, pos_right):
    lidx = left_idx.astype(jnp.int32)
    ridx = right_idx.astype(jnp.int32)
    k = pl.kernel(
        _body,
        out_type=jax.ShapeDtypeStruct((B_, D_), jnp.float32),
        mesh=plsc.VectorSubcoreMesh(core_axis_name="c", subcore_axis_name="s"),
        compiler_params=pltpu.CompilerParams(needs_layout_passes=False),
        scratch_types=(
            [pltpu.VMEM((ROWS_PER_W,), jnp.int32)] * 2
            + [pltpu.VMEM((MAXC, D_), jnp.float32)] * (NBUF + 2)
            + [pltpu.VMEM((D_,), jnp.float32)] * 2
            + [pltpu.SemaphoreType.DMA] * (NBUF + 3)
        ),
    )
    return k(lidx, ridx, class_emb, pos_left, pos_right)
